# baseline (speedup 1.0000x reference)
import jax
import jax.numpy as jnp
from jax.experimental import pallas as pl

_F32 = jnp.float32
_BF16 = jnp.bfloat16
_ROW_TILE = 256
_TOP_K = 16
_SEL_K = 8.0
_LAM = 0.5


def _bf(a):
    return a.astype(_BF16)


def _topo_tc_kernel(x_ref, W1T_ref, b1c_ref, W2T_ref, b2_ref, lt_ref,
                    mux_ref, sigx_ref, muk_ref, sigk_ref, mud_ref, sigd_ref,
                    Wlx_ref, wknn_ref, wden_ref, bl_ref, Wp_ref, bp_ref,
                    tokens_ref, y_ref):
    for b in range(x_ref.shape[0]):
        _one_batch(b, x_ref, W1T_ref, b1c_ref, W2T_ref, b2_ref, lt_ref,
                   mux_ref, sigx_ref, muk_ref, sigk_ref, mud_ref, sigd_ref,
                   Wlx_ref, wknn_ref, wden_ref, bl_ref, Wp_ref, bp_ref,
                   tokens_ref, y_ref)


def _one_batch(b, x_ref, W1T_ref, b1c_ref, W2T_ref, b2_ref, lt_ref,
               mux_ref, sigx_ref, muk_ref, sigk_ref, mud_ref, sigd_ref,
               Wlx_ref, wknn_ref, wden_ref, bl_ref, Wp_ref, bp_ref,
               tokens_ref, y_ref):
    x = x_ref[b]
    xT = x.T
    n = x.shape[0]
    rt = _ROW_TILE

    hT = jnp.maximum(
        jnp.dot(_bf(W1T_ref[...]), _bf(xT), preferred_element_type=_F32)
        + b1c_ref[...], 0.0)
    sal = (jnp.dot(_bf(W2T_ref[...]), _bf(hT), preferred_element_type=_F32)
           + b2_ref[0, 0])

    sq = jnp.sum(xT * xT, axis=0, keepdims=True)

    xT_bf = _bf(xT)
    iota_r = jax.lax.broadcasted_iota(jnp.int32, (rt, n), 0)
    iota_c = jax.lax.broadcasted_iota(jnp.int32, (rt, n), 1)
    dmin = jnp.full((1, n), 1e9, _F32)
    for t in range(n // rt):
        xt = x[t * rt:(t + 1) * rt]
        sqt = jnp.sum(xt * xt, axis=1, keepdims=True)
        g = jax.lax.dot_general(
            xT_bf[:, t * rt:(t + 1) * rt], xT_bf,
            (((0,), (0,)), ((), ())), preferred_element_type=_F32)
        d_t = jnp.maximum(sqt + sq - 2.0 * g, 0.0)
        d_t = d_t + jnp.where(iota_r + (t * rt) == iota_c, 1e9, 0.0)
        dmin = jnp.minimum(dmin, jnp.min(d_t, axis=0, keepdims=True))
    d_nn = jnp.sqrt(jnp.maximum(dmin, 0.0))
    density = 1.0 / (1.0 + d_nn)

    temp = jnp.clip(jnp.exp(lt_ref[0, 0]), 0.1, 10.0)
    logits = (sal / (2.0 * _LAM) - 0.5) / temp
    y = jax.nn.sigmoid(logits)
    budget = jnp.maximum(jnp.sum(y), 1e-6)
    y = y * jnp.minimum(_SEL_K / budget, 1.0)

    normv = jnp.sqrt(sq + d_nn * d_nn + density * density + sal * sal) + 1e-8
    fnT = jnp.concatenate(
        [xT / normv, d_nn / normv, density / normv, sal / normv], axis=0)
    fnT_bf = _bf(fnT)
    y_bf = _bf(y)
    chunks = []
    for t in range(n // rt):
        sim_t = jax.lax.dot_general(
            fnT_bf[:, t * rt:(t + 1) * rt], fnT_bf,
            (((0,), (0,)), ((), ())), preferred_element_type=_F32)
        chunks.append(jax.lax.dot_general(
            y_bf, _bf(sim_t), (((1,), (1,)), ((), ())),
            preferred_element_type=_F32))
    overlap = jnp.concatenate(chunks, axis=1)

    y = y / (1.0 + overlap)
    budget = jnp.maximum(jnp.sum(y), 1e-6)
    y_star = y * jnp.minimum(_SEL_K / budget, 1.0)
    y_ref[b] = y_star

    iota_1n = jax.lax.broadcasted_iota(jnp.int32, (1, n), 1)
    iota_k = jax.lax.broadcasted_iota(jnp.int32, (_TOP_K, 1), 0)

    def body(k, carry):
        yw, s = carry
        cur = jnp.max(yw)
        idx = jnp.min(jnp.where(yw == cur, iota_1n, n))
        hit = iota_1n == idx
        s = s + jnp.where(hit & (iota_k == k), 1.0, 0.0)
        yw = jnp.where(hit, -3.0e38, yw)
        return yw, s

    _, sel = jax.lax.fori_loop(
        0, _TOP_K, body, (y_star, jnp.zeros((_TOP_K, n), _F32)))

    sel_bf = _bf(sel)
    gx = jnp.dot(sel_bf, _bf(x), preferred_element_type=_F32)
    g_knn = jnp.sum(sel * d_nn, axis=1, keepdims=True)
    g_den = jnp.sum(sel * density, axis=1, keepdims=True)
    zx = (gx - mux_ref[...]) / sigx_ref[...]
    zk = (g_knn - muk_ref[0, 0]) / sigk_ref[0, 0]
    zd = (g_den - mud_ref[0, 0]) / sigd_ref[0, 0]
    pre = (jnp.dot(_bf(zx), _bf(Wlx_ref[...]), preferred_element_type=_F32)
           + _bf(zk).astype(_F32) * _bf(wknn_ref[...]).astype(_F32)
           + _bf(zd).astype(_F32) * _bf(wden_ref[...]).astype(_F32)
           + bl_ref[...])
    cloud = jnp.tanh(pre)
    tokens_ref[b] = (
        jnp.dot(_bf(cloud), _bf(Wp_ref[...]), preferred_element_type=_F32)
        + bp_ref[...])


def _specs(B, N, D, H):
    bcast = lambda shape: pl.BlockSpec(shape, lambda: tuple(0 for _ in shape))
    in_specs = [
        bcast((B, N, D)),
        bcast((H, D)),
        bcast((H, 1)),
        bcast((1, H)),
        bcast((1, 1)),
        bcast((1, 1)),
        bcast((1, D)),
        bcast((1, D)),
        bcast((1, 1)),
        bcast((1, 1)),
        bcast((1, 1)),
        bcast((1, 1)),
        bcast((D, _TOP_K)),
        bcast((1, _TOP_K)),
        bcast((1, _TOP_K)),
        bcast((1, _TOP_K)),
        bcast((_TOP_K, 256)),
        bcast((1, 256)),
    ]
    out_specs = (
        bcast((B, _TOP_K, 256)),
        bcast((B, 1, N)),
    )
    out_shape = (
        jax.ShapeDtypeStruct((B, _TOP_K, 256), _F32),
        jax.ShapeDtypeStruct((B, 1, N), _F32),
    )
    return in_specs, out_specs, out_shape


def _operands(x, W1, b1, W2, b2, log_temperature, mu, sigma, Wl, bl, Wp, bp):
    B, N, D = x.shape
    H = W1.shape[1]
    return (
        x,
        W1.T,
        b1.reshape(H, 1),
        W2.T,
        b2.reshape(1, 1),
        log_temperature.reshape(1, 1),
        mu[:D].reshape(1, D),
        sigma[:D].reshape(1, D),
        mu[D:D + 1].reshape(1, 1),
        sigma[D:D + 1].reshape(1, 1),
        mu[D + 1:D + 2].reshape(1, 1),
        sigma[D + 1:D + 2].reshape(1, 1),
        Wl[:D],
        Wl[D:D + 1],
        Wl[D + 1:D + 2],
        bl.reshape(1, -1),
        Wp,
        bp.reshape(1, -1),
    )


def kernel(x, W1, b1, W2, b2, log_temperature, mu, sigma, Wl, bl, Wp, bp):
    B, N, D = x.shape
    H = W1.shape[1]
    ops = _operands(x, W1, b1, W2, b2, log_temperature, mu, sigma,
                    Wl, bl, Wp, bp)
    in_specs, out_specs, out_shape = _specs(B, N, D, H)
    tokens, y2d = pl.pallas_call(
        _topo_tc_kernel,
        grid=(),
        in_specs=in_specs,
        out_specs=out_specs,
        out_shape=out_shape,
    )(*ops)
    return tokens, y2d.reshape(B, N)


# speedup vs baseline: 1.0583x; 1.0583x over previous
import jax
import jax.numpy as jnp
from jax.experimental import pallas as pl

_F32 = jnp.float32
_BF16 = jnp.bfloat16
_ROW_TILE = 256
_TOP_K = 16
_SEL_K = 8.0
_LAM = 0.5


def _bf(a):
    return a.astype(_BF16)


def _topo_tc_kernel(x_ref, W1T_ref, b1c_ref, W2T_ref, b2_ref, lt_ref,
                    mux_ref, sigx_ref, muk_ref, sigk_ref, mud_ref, sigd_ref,
                    Wlx_ref, wknn_ref, wden_ref, bl_ref, Wp_ref, bp_ref,
                    tokens_ref, y_ref):
    for b in range(x_ref.shape[0]):
        _one_batch(b, x_ref, W1T_ref, b1c_ref, W2T_ref, b2_ref, lt_ref,
                   mux_ref, sigx_ref, muk_ref, sigk_ref, mud_ref, sigd_ref,
                   Wlx_ref, wknn_ref, wden_ref, bl_ref, Wp_ref, bp_ref,
                   tokens_ref, y_ref)


def _one_batch(b, x_ref, W1T_ref, b1c_ref, W2T_ref, b2_ref, lt_ref,
               mux_ref, sigx_ref, muk_ref, sigk_ref, mud_ref, sigd_ref,
               Wlx_ref, wknn_ref, wden_ref, bl_ref, Wp_ref, bp_ref,
               tokens_ref, y_ref):
    x = x_ref[b]
    xT = x.T
    n = x.shape[0]
    rt = _ROW_TILE

    hT = jnp.maximum(
        jnp.dot(_bf(W1T_ref[...]), _bf(xT), preferred_element_type=_F32)
        + b1c_ref[...], 0.0)
    sal = (jnp.dot(_bf(W2T_ref[...]), _bf(hT), preferred_element_type=_F32)
           + b2_ref[0, 0])

    sq = jnp.sum(xT * xT, axis=0, keepdims=True)

    xT_bf = _bf(xT)
    xT2_bf = _bf(xT + xT)
    iota_r = jax.lax.broadcasted_iota(jnp.int32, (rt, rt), 0)
    iota_c = jax.lax.broadcasted_iota(jnp.int32, (rt, rt), 1)
    eyemask = jnp.where(iota_r == iota_c, 1e9, 0.0)
    part = jnp.full((1, n), 1e9, _F32)
    for t in range(n // rt):
        xt = x[t * rt:(t + 1) * rt]
        sqt = jnp.sum(xt * xt, axis=1, keepdims=True)
        g2 = jax.lax.dot_general(
            xT2_bf[:, t * rt:(t + 1) * rt], xT_bf,
            (((0,), (0,)), ((), ())), preferred_element_type=_F32)
        tmp = sqt - g2
        m_all = jnp.min(tmp, axis=0, keepdims=True)
        m_diag = jnp.min(tmp[:, t * rt:(t + 1) * rt] + eyemask,
                         axis=0, keepdims=True)
        pieces = [m_diag] if t == 0 else [m_all[:, :t * rt], m_diag]
        if t < n // rt - 1:
            pieces.append(m_all[:, (t + 1) * rt:])
        part = jnp.minimum(part, jnp.concatenate(pieces, axis=1))
    d_nn = jnp.sqrt(jnp.maximum(part + sq, 0.0))
    density = 1.0 / (1.0 + d_nn)

    temp = jnp.clip(jnp.exp(lt_ref[0, 0]), 0.1, 10.0)
    logits = (sal / (2.0 * _LAM) - 0.5) / temp
    y = jax.nn.sigmoid(logits)
    budget = jnp.maximum(jnp.sum(y), 1e-6)
    y = y * jnp.minimum(_SEL_K / budget, 1.0)

    normv = jnp.sqrt(sq + d_nn * d_nn + density * density + sal * sal) + 1e-8
    fnT = jnp.concatenate(
        [xT / normv, d_nn / normv, density / normv, sal / normv], axis=0)
    fnT_bf = _bf(fnT)
    y_bf = _bf(y)
    chunks = []
    for t in range(n // rt):
        sim_t = jax.lax.dot_general(
            fnT_bf[:, t * rt:(t + 1) * rt], fnT_bf,
            (((0,), (0,)), ((), ())), preferred_element_type=_F32)
        chunks.append(jax.lax.dot_general(
            y_bf, _bf(sim_t), (((1,), (1,)), ((), ())),
            preferred_element_type=_F32))
    overlap = jnp.concatenate(chunks, axis=1)

    y = y / (1.0 + overlap)
    budget = jnp.maximum(jnp.sum(y), 1e-6)
    y_star = y * jnp.minimum(_SEL_K / budget, 1.0)
    y_ref[b] = y_star

    iota_1n = jax.lax.broadcasted_iota(jnp.int32, (1, n), 1)
    iota_k = jax.lax.broadcasted_iota(jnp.int32, (_TOP_K, 1), 0)

    def body(k, carry):
        yw, s = carry
        cur = jnp.max(yw)
        idx = jnp.min(jnp.where(yw == cur, iota_1n, n))
        hit = iota_1n == idx
        s = s + jnp.where(hit & (iota_k == k), 1.0, 0.0)
        yw = jnp.where(hit, -3.0e38, yw)
        return yw, s

    _, sel = jax.lax.fori_loop(
        0, _TOP_K, body, (y_star, jnp.zeros((_TOP_K, n), _F32)))

    sel_bf = _bf(sel)
    gx = jnp.dot(sel_bf, _bf(x), preferred_element_type=_F32)
    g_knn = jnp.sum(sel * d_nn, axis=1, keepdims=True)
    g_den = jnp.sum(sel * density, axis=1, keepdims=True)
    zx = (gx - mux_ref[...]) / sigx_ref[...]
    zk = (g_knn - muk_ref[0, 0]) / sigk_ref[0, 0]
    zd = (g_den - mud_ref[0, 0]) / sigd_ref[0, 0]
    pre = (jnp.dot(_bf(zx), _bf(Wlx_ref[...]), preferred_element_type=_F32)
           + _bf(zk).astype(_F32) * _bf(wknn_ref[...]).astype(_F32)
           + _bf(zd).astype(_F32) * _bf(wden_ref[...]).astype(_F32)
           + bl_ref[...])
    cloud = jnp.tanh(pre)
    tokens_ref[b] = (
        jnp.dot(_bf(cloud), _bf(Wp_ref[...]), preferred_element_type=_F32)
        + bp_ref[...])


def _specs(B, N, D, H):
    bcast = lambda shape: pl.BlockSpec(shape, lambda: tuple(0 for _ in shape))
    in_specs = [
        bcast((B, N, D)),
        bcast((H, D)),
        bcast((H, 1)),
        bcast((1, H)),
        bcast((1, 1)),
        bcast((1, 1)),
        bcast((1, D)),
        bcast((1, D)),
        bcast((1, 1)),
        bcast((1, 1)),
        bcast((1, 1)),
        bcast((1, 1)),
        bcast((D, _TOP_K)),
        bcast((1, _TOP_K)),
        bcast((1, _TOP_K)),
        bcast((1, _TOP_K)),
        bcast((_TOP_K, 256)),
        bcast((1, 256)),
    ]
    out_specs = (
        bcast((B, _TOP_K, 256)),
        bcast((B, 1, N)),
    )
    out_shape = (
        jax.ShapeDtypeStruct((B, _TOP_K, 256), _F32),
        jax.ShapeDtypeStruct((B, 1, N), _F32),
    )
    return in_specs, out_specs, out_shape


def _operands(x, W1, b1, W2, b2, log_temperature, mu, sigma, Wl, bl, Wp, bp):
    B, N, D = x.shape
    H = W1.shape[1]
    return (
        x,
        W1.T,
        b1.reshape(H, 1),
        W2.T,
        b2.reshape(1, 1),
        log_temperature.reshape(1, 1),
        mu[:D].reshape(1, D),
        sigma[:D].reshape(1, D),
        mu[D:D + 1].reshape(1, 1),
        sigma[D:D + 1].reshape(1, 1),
        mu[D + 1:D + 2].reshape(1, 1),
        sigma[D + 1:D + 2].reshape(1, 1),
        Wl[:D],
        Wl[D:D + 1],
        Wl[D + 1:D + 2],
        bl.reshape(1, -1),
        Wp,
        bp.reshape(1, -1),
    )


def kernel(x, W1, b1, W2, b2, log_temperature, mu, sigma, Wl, bl, Wp, bp):
    B, N, D = x.shape
    H = W1.shape[1]
    ops = _operands(x, W1, b1, W2, b2, log_temperature, mu, sigma,
                    Wl, bl, Wp, bp)
    in_specs, out_specs, out_shape = _specs(B, N, D, H)
    tokens, y2d = pl.pallas_call(
        _topo_tc_kernel,
        grid=(),
        in_specs=in_specs,
        out_specs=out_specs,
        out_shape=out_shape,
    )(*ops)
    return tokens, y2d.reshape(B, N)


# speedup vs baseline: 1.1440x; 1.0811x over previous
import jax
import jax.numpy as jnp
from jax.experimental import pallas as pl

_F32 = jnp.float32
_BF16 = jnp.bfloat16
_ROW_TILE = 256
_TOP_K = 16
_SEL_K = 8.0
_LAM = 0.5


def _bf(a):
    return a.astype(_BF16)


def _topo_tc_kernel(x_ref, W1T_ref, b1c_ref, W2T_ref, b2_ref, lt_ref,
                    mux_ref, sigx_ref, muk_ref, sigk_ref, mud_ref, sigd_ref,
                    Wlx_ref, wknn_ref, wden_ref, bl_ref, Wp_ref, bp_ref,
                    tokens_ref, y_ref):
    for b in range(x_ref.shape[0]):
        _one_batch(b, x_ref, W1T_ref, b1c_ref, W2T_ref, b2_ref, lt_ref,
                   mux_ref, sigx_ref, muk_ref, sigk_ref, mud_ref, sigd_ref,
                   Wlx_ref, wknn_ref, wden_ref, bl_ref, Wp_ref, bp_ref,
                   tokens_ref, y_ref)


def _one_batch(b, x_ref, W1T_ref, b1c_ref, W2T_ref, b2_ref, lt_ref,
               mux_ref, sigx_ref, muk_ref, sigk_ref, mud_ref, sigd_ref,
               Wlx_ref, wknn_ref, wden_ref, bl_ref, Wp_ref, bp_ref,
               tokens_ref, y_ref):
    x = x_ref[b]
    xT = x.T
    n = x.shape[0]
    rt = _ROW_TILE

    hT = jnp.maximum(
        jnp.dot(_bf(W1T_ref[...]), _bf(xT), preferred_element_type=_F32)
        + b1c_ref[...], 0.0)
    sal = (jnp.dot(_bf(W2T_ref[...]), _bf(hT), preferred_element_type=_F32)
           + b2_ref[0, 0])

    sq = jnp.sum(xT * xT, axis=0, keepdims=True)

    xT_bf = _bf(xT)
    xT2_bf = _bf(xT + xT)
    iota_r = jax.lax.broadcasted_iota(jnp.int32, (rt, rt), 0)
    iota_c = jax.lax.broadcasted_iota(jnp.int32, (rt, rt), 1)
    eyemask = jnp.where(iota_r == iota_c, 1e9, 0.0)
    nt = n // rt
    part_cols = jnp.full((1, n), 1e9, _F32)
    row_chunks = []
    for t in range(nt):
        lo = t * rt
        xt = x[lo:lo + rt]
        sqt = jnp.sum(xt * xt, axis=1, keepdims=True)
        g2 = jax.lax.dot_general(
            xT2_bf[:, lo:lo + rt], xT_bf[:, lo:],
            (((0,), (0,)), ((), ())), preferred_element_type=_F32)
        ctmp = sqt - g2
        m_diag = jnp.min(ctmp[:, :rt] + eyemask, axis=0, keepdims=True)
        if t < nt - 1:
            m_rest = jnp.min(ctmp[:, rt:], axis=0, keepdims=True)
            mrow = jnp.concatenate([m_diag, m_rest], axis=1)
        else:
            mrow = m_diag
        if t > 0:
            mrow = jnp.concatenate(
                [jnp.full((1, lo), 1e9, _F32), mrow], axis=1)
        part_cols = jnp.minimum(part_cols, mrow)
        rtmp = sq[:, lo:] - g2
        rmin = jnp.min(rtmp[:, :rt] + eyemask, axis=1, keepdims=True)
        if t < nt - 1:
            rmin = jnp.minimum(
                rmin, jnp.min(rtmp[:, rt:], axis=1, keepdims=True))
        row_chunks.append(rmin.T)
    part = jnp.minimum(part_cols, jnp.concatenate(row_chunks, axis=1))
    d_nn = jnp.sqrt(jnp.maximum(part + sq, 0.0))
    density = 1.0 / (1.0 + d_nn)

    temp = jnp.clip(jnp.exp(lt_ref[0, 0]), 0.1, 10.0)
    logits = (sal / (2.0 * _LAM) - 0.5) / temp
    y = jax.nn.sigmoid(logits)
    budget = jnp.maximum(jnp.sum(y), 1e-6)
    y = y * jnp.minimum(_SEL_K / budget, 1.0)

    normv = jnp.sqrt(sq + d_nn * d_nn + density * density + sal * sal) + 1e-8
    fnT = jnp.concatenate(
        [xT / normv, d_nn / normv, density / normv, sal / normv], axis=0)
    fnT_bf = _bf(fnT)
    y_bf = _bf(y)
    row_ov = []
    acc = jnp.zeros((1, n), _F32)
    for t in range(nt):
        lo = t * rt
        sim_t = jax.lax.dot_general(
            fnT_bf[:, lo:lo + rt], fnT_bf[:, lo:],
            (((0,), (0,)), ((), ())), preferred_element_type=_F32)
        sim_bf = _bf(sim_t)
        row_ov.append(jax.lax.dot_general(
            y_bf[:, lo:], sim_bf, (((1,), (1,)), ((), ())),
            preferred_element_type=_F32))
        if t < nt - 1:
            colc = jax.lax.dot_general(
                y_bf[:, lo:lo + rt], sim_bf[:, rt:],
                (((1,), (0,)), ((), ())),
                preferred_element_type=_F32)
            acc = acc + jnp.concatenate(
                [jnp.zeros((1, lo + rt), _F32), colc], axis=1)
    overlap = jnp.concatenate(row_ov, axis=1) + acc

    y = y / (1.0 + overlap)
    budget = jnp.maximum(jnp.sum(y), 1e-6)
    y_star = y * jnp.minimum(_SEL_K / budget, 1.0)
    y_ref[b] = y_star

    iota_1n = jax.lax.broadcasted_iota(jnp.int32, (1, n), 1)
    iota_k = jax.lax.broadcasted_iota(jnp.int32, (_TOP_K, 1), 0)

    def body(k, carry):
        yw, s = carry
        cur = jnp.max(yw)
        idx = jnp.min(jnp.where(yw == cur, iota_1n, n))
        hit = iota_1n == idx
        s = s + jnp.where(hit & (iota_k == k), 1.0, 0.0)
        yw = jnp.where(hit, -3.0e38, yw)
        return yw, s

    _, sel = jax.lax.fori_loop(
        0, _TOP_K, body, (y_star, jnp.zeros((_TOP_K, n), _F32)))

    sel_bf = _bf(sel)
    gx = jnp.dot(sel_bf, _bf(x), preferred_element_type=_F32)
    g_knn = jnp.sum(sel * d_nn, axis=1, keepdims=True)
    g_den = jnp.sum(sel * density, axis=1, keepdims=True)
    zx = (gx - mux_ref[...]) / sigx_ref[...]
    zk = (g_knn - muk_ref[0, 0]) / sigk_ref[0, 0]
    zd = (g_den - mud_ref[0, 0]) / sigd_ref[0, 0]
    pre = (jnp.dot(_bf(zx), _bf(Wlx_ref[...]), preferred_element_type=_F32)
           + _bf(zk).astype(_F32) * _bf(wknn_ref[...]).astype(_F32)
           + _bf(zd).astype(_F32) * _bf(wden_ref[...]).astype(_F32)
           + bl_ref[...])
    cloud = jnp.tanh(pre)
    tokens_ref[b] = (
        jnp.dot(_bf(cloud), _bf(Wp_ref[...]), preferred_element_type=_F32)
        + bp_ref[...])


def _specs(B, N, D, H):
    bcast = lambda shape: pl.BlockSpec(shape, lambda: tuple(0 for _ in shape))
    in_specs = [
        bcast((B, N, D)),
        bcast((H, D)),
        bcast((H, 1)),
        bcast((1, H)),
        bcast((1, 1)),
        bcast((1, 1)),
        bcast((1, D)),
        bcast((1, D)),
        bcast((1, 1)),
        bcast((1, 1)),
        bcast((1, 1)),
        bcast((1, 1)),
        bcast((D, _TOP_K)),
        bcast((1, _TOP_K)),
        bcast((1, _TOP_K)),
        bcast((1, _TOP_K)),
        bcast((_TOP_K, 256)),
        bcast((1, 256)),
    ]
    out_specs = (
        bcast((B, _TOP_K, 256)),
        bcast((B, 1, N)),
    )
    out_shape = (
        jax.ShapeDtypeStruct((B, _TOP_K, 256), _F32),
        jax.ShapeDtypeStruct((B, 1, N), _F32),
    )
    return in_specs, out_specs, out_shape


def _operands(x, W1, b1, W2, b2, log_temperature, mu, sigma, Wl, bl, Wp, bp):
    B, N, D = x.shape
    H = W1.shape[1]
    return (
        x,
        W1.T,
        b1.reshape(H, 1),
        W2.T,
        b2.reshape(1, 1),
        log_temperature.reshape(1, 1),
        mu[:D].reshape(1, D),
        sigma[:D].reshape(1, D),
        mu[D:D + 1].reshape(1, 1),
        sigma[D:D + 1].reshape(1, 1),
        mu[D + 1:D + 2].reshape(1, 1),
        sigma[D + 1:D + 2].reshape(1, 1),
        Wl[:D],
        Wl[D:D + 1],
        Wl[D + 1:D + 2],
        bl.reshape(1, -1),
        Wp,
        bp.reshape(1, -1),
    )


def kernel(x, W1, b1, W2, b2, log_temperature, mu, sigma, Wl, bl, Wp, bp):
    B, N, D = x.shape
    H = W1.shape[1]
    ops = _operands(x, W1, b1, W2, b2, log_temperature, mu, sigma,
                    Wl, bl, Wp, bp)
    in_specs, out_specs, out_shape = _specs(B, N, D, H)
    tokens, y2d = pl.pallas_call(
        _topo_tc_kernel,
        grid=(),
        in_specs=in_specs,
        out_specs=out_specs,
        out_shape=out_shape,
    )(*ops)
    return tokens, y2d.reshape(B, N)


# speedup vs baseline: 1.1598x; 1.0138x over previous
import jax
import jax.numpy as jnp
from jax.experimental import pallas as pl

_F32 = jnp.float32
_BF16 = jnp.bfloat16
_ROW_TILE = 512
_TOP_K = 16
_SEL_K = 8.0
_LAM = 0.5


def _bf(a):
    return a.astype(_BF16)


def _topo_tc_kernel(x_ref, W1T_ref, b1c_ref, W2T_ref, b2_ref, lt_ref,
                    mux_ref, sigx_ref, muk_ref, sigk_ref, mud_ref, sigd_ref,
                    Wlx_ref, wknn_ref, wden_ref, bl_ref, Wp_ref, bp_ref,
                    tokens_ref, y_ref):
    for b in range(x_ref.shape[0]):
        _one_batch(b, x_ref, W1T_ref, b1c_ref, W2T_ref, b2_ref, lt_ref,
                   mux_ref, sigx_ref, muk_ref, sigk_ref, mud_ref, sigd_ref,
                   Wlx_ref, wknn_ref, wden_ref, bl_ref, Wp_ref, bp_ref,
                   tokens_ref, y_ref)


def _one_batch(b, x_ref, W1T_ref, b1c_ref, W2T_ref, b2_ref, lt_ref,
               mux_ref, sigx_ref, muk_ref, sigk_ref, mud_ref, sigd_ref,
               Wlx_ref, wknn_ref, wden_ref, bl_ref, Wp_ref, bp_ref,
               tokens_ref, y_ref):
    x = x_ref[b]
    xT = x.T
    n = x.shape[0]
    rt = _ROW_TILE

    hT = jnp.maximum(
        jnp.dot(_bf(W1T_ref[...]), _bf(xT), preferred_element_type=_F32)
        + b1c_ref[...], 0.0)
    sal = (jnp.dot(_bf(W2T_ref[...]), _bf(hT), preferred_element_type=_F32)
           + b2_ref[0, 0])

    sq = jnp.sum(xT * xT, axis=0, keepdims=True)

    xT_bf = _bf(xT)
    xT2_bf = _bf(xT + xT)
    iota_r = jax.lax.broadcasted_iota(jnp.int32, (rt, rt), 0)
    iota_c = jax.lax.broadcasted_iota(jnp.int32, (rt, rt), 1)
    eyemask = jnp.where(iota_r == iota_c, 1e9, 0.0)
    nt = n // rt
    part_cols = jnp.full((1, n), 1e9, _F32)
    row_chunks = []
    for t in range(nt):
        lo = t * rt
        xt = x[lo:lo + rt]
        sqt = jnp.sum(xt * xt, axis=1, keepdims=True)
        g2 = jax.lax.dot_general(
            xT2_bf[:, lo:lo + rt], xT_bf[:, lo:],
            (((0,), (0,)), ((), ())), preferred_element_type=_F32)
        ctmp = sqt - g2
        m_diag = jnp.min(ctmp[:, :rt] + eyemask, axis=0, keepdims=True)
        if t < nt - 1:
            m_rest = jnp.min(ctmp[:, rt:], axis=0, keepdims=True)
            mrow = jnp.concatenate([m_diag, m_rest], axis=1)
        else:
            mrow = m_diag
        if t > 0:
            mrow = jnp.concatenate(
                [jnp.full((1, lo), 1e9, _F32), mrow], axis=1)
        part_cols = jnp.minimum(part_cols, mrow)
        rtmp = sq[:, lo:] - g2
        rmin = jnp.min(rtmp[:, :rt] + eyemask, axis=1, keepdims=True)
        if t < nt - 1:
            rmin = jnp.minimum(
                rmin, jnp.min(rtmp[:, rt:], axis=1, keepdims=True))
        row_chunks.append(rmin.T)
    part = jnp.minimum(part_cols, jnp.concatenate(row_chunks, axis=1))
    d_nn = jnp.sqrt(jnp.maximum(part + sq, 0.0))
    density = 1.0 / (1.0 + d_nn)

    temp = jnp.clip(jnp.exp(lt_ref[0, 0]), 0.1, 10.0)
    logits = (sal / (2.0 * _LAM) - 0.5) / temp
    y = jax.nn.sigmoid(logits)
    budget = jnp.maximum(jnp.sum(y), 1e-6)
    y = y * jnp.minimum(_SEL_K / budget, 1.0)

    normv = jnp.sqrt(sq + d_nn * d_nn + density * density + sal * sal) + 1e-8
    fnT = jnp.concatenate(
        [xT / normv, d_nn / normv, density / normv, sal / normv], axis=0)
    fnT_bf = _bf(fnT)
    y_bf = _bf(y)
    row_ov = []
    acc = jnp.zeros((1, n), _F32)
    for t in range(nt):
        lo = t * rt
        sim_t = jax.lax.dot_general(
            fnT_bf[:, lo:lo + rt], fnT_bf[:, lo:],
            (((0,), (0,)), ((), ())), preferred_element_type=_F32)
        sim_bf = _bf(sim_t)
        row_ov.append(jax.lax.dot_general(
            y_bf[:, lo:], sim_bf, (((1,), (1,)), ((), ())),
            preferred_element_type=_F32))
        if t < nt - 1:
            colc = jax.lax.dot_general(
                y_bf[:, lo:lo + rt], sim_bf[:, rt:],
                (((1,), (0,)), ((), ())),
                preferred_element_type=_F32)
            acc = acc + jnp.concatenate(
                [jnp.zeros((1, lo + rt), _F32), colc], axis=1)
    overlap = jnp.concatenate(row_ov, axis=1) + acc

    y = y / (1.0 + overlap)
    budget = jnp.maximum(jnp.sum(y), 1e-6)
    y_star = y * jnp.minimum(_SEL_K / budget, 1.0)
    y_ref[b] = y_star

    iota_1n = jax.lax.broadcasted_iota(jnp.int32, (1, n), 1)
    iota_k = jax.lax.broadcasted_iota(jnp.int32, (_TOP_K, 1), 0)

    def body(k, carry):
        yw, s = carry
        cur = jnp.max(yw)
        idx = jnp.min(jnp.where(yw == cur, iota_1n, n))
        hit = iota_1n == idx
        s = s + jnp.where(hit & (iota_k == k), 1.0, 0.0)
        yw = jnp.where(hit, -3.0e38, yw)
        return yw, s

    _, sel = jax.lax.fori_loop(
        0, _TOP_K, body, (y_star, jnp.zeros((_TOP_K, n), _F32)))

    sel_bf = _bf(sel)
    gx = jnp.dot(sel_bf, _bf(x), preferred_element_type=_F32)
    g_knn = jnp.sum(sel * d_nn, axis=1, keepdims=True)
    g_den = jnp.sum(sel * density, axis=1, keepdims=True)
    zx = (gx - mux_ref[...]) / sigx_ref[...]
    zk = (g_knn - muk_ref[0, 0]) / sigk_ref[0, 0]
    zd = (g_den - mud_ref[0, 0]) / sigd_ref[0, 0]
    pre = (jnp.dot(_bf(zx), _bf(Wlx_ref[...]), preferred_element_type=_F32)
           + _bf(zk).astype(_F32) * _bf(wknn_ref[...]).astype(_F32)
           + _bf(zd).astype(_F32) * _bf(wden_ref[...]).astype(_F32)
           + bl_ref[...])
    cloud = jnp.tanh(pre)
    tokens_ref[b] = (
        jnp.dot(_bf(cloud), _bf(Wp_ref[...]), preferred_element_type=_F32)
        + bp_ref[...])


def _specs(B, N, D, H):
    bcast = lambda shape: pl.BlockSpec(shape, lambda: tuple(0 for _ in shape))
    in_specs = [
        bcast((B, N, D)),
        bcast((H, D)),
        bcast((H, 1)),
        bcast((1, H)),
        bcast((1, 1)),
        bcast((1, 1)),
        bcast((1, D)),
        bcast((1, D)),
        bcast((1, 1)),
        bcast((1, 1)),
        bcast((1, 1)),
        bcast((1, 1)),
        bcast((D, _TOP_K)),
        bcast((1, _TOP_K)),
        bcast((1, _TOP_K)),
        bcast((1, _TOP_K)),
        bcast((_TOP_K, 256)),
        bcast((1, 256)),
    ]
    out_specs = (
        bcast((B, _TOP_K, 256)),
        bcast((B, 1, N)),
    )
    out_shape = (
        jax.ShapeDtypeStruct((B, _TOP_K, 256), _F32),
        jax.ShapeDtypeStruct((B, 1, N), _F32),
    )
    return in_specs, out_specs, out_shape


def _operands(x, W1, b1, W2, b2, log_temperature, mu, sigma, Wl, bl, Wp, bp):
    B, N, D = x.shape
    H = W1.shape[1]
    return (
        x,
        W1.T,
        b1.reshape(H, 1),
        W2.T,
        b2.reshape(1, 1),
        log_temperature.reshape(1, 1),
        mu[:D].reshape(1, D),
        sigma[:D].reshape(1, D),
        mu[D:D + 1].reshape(1, 1),
        sigma[D:D + 1].reshape(1, 1),
        mu[D + 1:D + 2].reshape(1, 1),
        sigma[D + 1:D + 2].reshape(1, 1),
        Wl[:D],
        Wl[D:D + 1],
        Wl[D + 1:D + 2],
        bl.reshape(1, -1),
        Wp,
        bp.reshape(1, -1),
    )


def kernel(x, W1, b1, W2, b2, log_temperature, mu, sigma, Wl, bl, Wp, bp):
    B, N, D = x.shape
    H = W1.shape[1]
    ops = _operands(x, W1, b1, W2, b2, log_temperature, mu, sigma,
                    Wl, bl, Wp, bp)
    in_specs, out_specs, out_shape = _specs(B, N, D, H)
    tokens, y2d = pl.pallas_call(
        _topo_tc_kernel,
        grid=(),
        in_specs=in_specs,
        out_specs=out_specs,
        out_shape=out_shape,
    )(*ops)
    return tokens, y2d.reshape(B, N)
